# baseline (speedup 1.0000x reference)
import jax
import jax.numpy as jnp
from jax import lax
from jax.experimental import pallas as pl
from jax.experimental.pallas import tpu as pltpu
from jax.experimental.pallas import tpu_sc as plsc

NC, NS, LANES = 2, 16, 16
NW = NC * NS
CHUNK = 64
NGP = CHUNK // LANES
PACKW = 3 * CHUNK


def _project_linf(W, v):
    m = W.shape[1]
    absW = jnp.abs(W)
    s = jnp.sum(absW, axis=1, keepdims=True)
    u = jnp.sort(absW, axis=1)[:, ::-1]
    css = jnp.cumsum(u, axis=1)
    idx = jnp.arange(1, m + 1)
    cond = u - (css - v) / idx.astype(W.dtype) > 0
    rho = jnp.max(jnp.where(cond, idx, 0), axis=1, keepdims=True)
    theta = (jnp.take_along_axis(css, rho - 1, axis=1) - v) / rho.astype(W.dtype)
    proj = jnp.sign(W) * jnp.maximum(absW - theta, 0.0)
    return jnp.where(s > v, proj, W)


def _mm_body(m_ref, w_ref, o_ref):
    o_ref[...] = jnp.dot(m_ref[...], w_ref[...], preferred_element_type=jnp.float32)


def _tc_matmul(M, Wt, blk):
    n_pad, m = M.shape
    return pl.pallas_call(
        _mm_body,
        grid=(n_pad // blk,),
        in_specs=[
            pl.BlockSpec((blk, m), lambda i: (i, 0)),
            pl.BlockSpec((m, m), lambda i: (0, 0)),
        ],
        out_specs=pl.BlockSpec((blk, m), lambda i: (i, 0)),
        out_shape=jax.ShapeDtypeStruct((n_pad, m), jnp.float32),
    )(M, Wt)


def _sload(ref, i):
    return ref[pl.ds(i, LANES)][0]


def _make_seg_kernel(n_pad, m, rows, with_bias):
    mesh = plsc.VectorSubcoreMesh(
        core_axis_name="c", subcore_axis_name="s", num_cores=NC, num_subcores=NS
    )
    acc_words = rows * m

    def body(y_hbm, pack_hbm, meta_hbm, *rest):
        if with_bias:
            (bias_hbm, out_hbm, acc, gb0, gb1, ib0, ib1, metav,
             sg0, sg1, si0, si1) = rest
        else:
            (out_hbm, acc, gb0, gb1, ib0, ib1, metav,
             sg0, sg1, si0, si1) = rest
        c_id = lax.axis_index("c")
        s_id = lax.axis_index("s")
        wid = c_id * NS + s_id
        base_row = wid * rows
        iota = jnp.arange(LANES, dtype=jnp.int32)

        pltpu.sync_copy(meta_hbm, metav)
        start = _sload(metav, wid)
        end = _sload(metav, NW + wid)
        start_row = start // CHUNK
        nch = (end - start_row * CHUNK + (CHUNK - 1)) // CHUNK
        nblk2 = jnp.maximum((nch + 1) // 2, 1)

        def fire_idx(row, ib, sem):
            pltpu.async_copy(pack_hbm.at[pl.ds(row, 1)], ib, sem)

        def wait_idx(ib, sem):
            pltpu.make_async_copy(pack_hbm.at[pl.ds(0, 1)], ib, sem).wait()

        def fire_g(ib, gb, sem):
            pltpu.async_copy(y_hbm.at[ib.at[0, pl.ds(0, CHUNK)]], gb, sem)

        def wait_g(ib, gb, sem):
            pltpu.make_async_copy(y_hbm.at[ib.at[0, pl.ds(0, CHUNK)]], gb, sem).wait()

        if with_bias:
            pltpu.sync_copy(bias_hbm.at[pl.ds(base_row * m, acc_words)], acc)
        else:
            zeros16 = jnp.zeros((LANES,), jnp.float32)

            def zero_body(i, _):
                acc[pl.ds(i * LANES, LANES)] = zeros16
                return 0

            lax.fori_loop(0, acc_words // LANES, zero_body, 0)

        def process(off, ib, gb):
            def grp(gi):
                lo = gi * LANES
                eidv = off + lo + iota
                valid = (eidv >= start) & (eidv < end)
                d16 = jnp.where(valid, ib[0, pl.ds(CHUNK + lo, LANES)] - base_row, 0)
                w16 = jnp.where(
                    valid,
                    plsc.bitcast(ib[0, pl.ds(2 * CHUNK + lo, LANES)], jnp.float32),
                    0.0,
                )
                a16 = d16 * m
                for i in range(LANES):
                    w1 = jnp.broadcast_to(w16[i], (LANES,))
                    ab = a16[i]
                    eb = lo + i
                    for j in range(m // LANES):
                        x = gb[eb, pl.ds(j * LANES, LANES)]
                        plsc.addupdate(acc.at[pl.ds(ab + j * LANES, LANES)], x * w1)

            plsc.parallel_loop(0, NGP, 1, unroll=1)(grp)

        fire_idx(start_row, ib0, si0)
        wait_idx(ib0, si0)
        fire_g(ib0, gb0, sg0)
        fire_idx(start_row + 1, ib1, si1)

        def blk_body(bi, _):
            for b, (ib_c, gb_c, si_c, sg_c, ib_n, gb_n, si_n, sg_n) in enumerate(
                ((ib0, gb0, si0, sg0, ib1, gb1, si1, sg1),
                 (ib1, gb1, si1, sg1, ib0, gb0, si0, sg0))
            ):
                cc = bi * 2 + b
                wait_g(ib_c, gb_c, sg_c)
                wait_idx(ib_n, si_n)
                fire_g(ib_n, gb_n, sg_n)
                process((start_row + cc) * CHUNK, ib_c, gb_c)
                fire_idx(start_row + cc + 2, ib_c, si_c)
            return 0

        lax.fori_loop(0, nblk2, blk_body, 0)

        wait_g(ib0, gb0, sg0)
        wait_idx(ib1, si1)

        if with_bias:
            def relu_body(i, _):
                v = acc[pl.ds(i * LANES, LANES)]
                acc[pl.ds(i * LANES, LANES)] = jnp.maximum(v, 0.0)
                return 0

            lax.fori_loop(0, acc_words // LANES, relu_body, 0)

        pltpu.sync_copy(acc, out_hbm.at[pl.ds(base_row * m, acc_words)])

    scratch = [
        pltpu.VMEM((acc_words,), jnp.float32),
        pltpu.VMEM((CHUNK, m), jnp.float32),
        pltpu.VMEM((CHUNK, m), jnp.float32),
        pltpu.VMEM((1, PACKW), jnp.int32),
        pltpu.VMEM((1, PACKW), jnp.int32),
        pltpu.VMEM((NW * 2 + LANES,), jnp.int32),
        pltpu.SemaphoreType.DMA,
        pltpu.SemaphoreType.DMA,
        pltpu.SemaphoreType.DMA,
        pltpu.SemaphoreType.DMA,
    ]
    return pl.kernel(
        body,
        out_type=jax.ShapeDtypeStruct((n_pad * m,), jnp.float32),
        mesh=mesh,
        scratch_types=scratch,
        compiler_params=pltpu.CompilerParams(
            use_tc_tiling_on_sc=False, needs_layout_passes=False
        ),
    )


def kernel(X_0, edge_index, edge_weight, U, W, Omega_1, fw_mitr):
    m, n = X_0.shape
    p = U.shape[0]
    E = edge_index.shape[1]
    kappa, A_rho = 0.99, 1.0

    rows = ((n + NW - 1) // NW + 7) // 8 * 8
    n_pad = NW * rows
    e_pad = ((E + CHUNK - 1) // CHUNK) * CHUNK + 8 * CHUNK

    W_p = _project_linf(W, kappa / A_rho)
    Wt = W_p.T

    src = edge_index[0].astype(jnp.int32)
    dst = edge_index[1].astype(jnp.int32)
    order = jnp.argsort(dst)
    src_s = jnp.concatenate([src[order], jnp.zeros((e_pad - E,), jnp.int32)])
    dst_s = jnp.concatenate([dst[order], jnp.zeros((e_pad - E,), jnp.int32)])
    w_s = jnp.concatenate(
        [edge_weight[order].astype(jnp.float32), jnp.zeros((e_pad - E,), jnp.float32)]
    )
    nrow = e_pad // CHUNK
    pack = jnp.concatenate(
        [
            src_s.reshape(nrow, CHUNK),
            dst_s.reshape(nrow, CHUNK),
            lax.bitcast_convert_type(w_s, jnp.int32).reshape(nrow, CHUNK),
        ],
        axis=1,
    )
    bounds = jnp.searchsorted(
        dst_s[:E], jnp.arange(NW + 1, dtype=jnp.int32) * rows
    ).astype(jnp.int32)
    meta = jnp.concatenate(
        [bounds[:NW], bounds[1 : NW + 1], jnp.zeros((LANES,), jnp.int32)]
    )

    seg_plain = _make_seg_kernel(n_pad, m, rows, with_bias=False)
    seg_bias_relu = _make_seg_kernel(n_pad, m, rows, with_bias=True)

    ut_pad = jnp.zeros((n_pad, p), jnp.float32).at[:n].set(U.T)
    s1_nm = _tc_matmul(ut_pad, Omega_1.T, blk=1024)
    b_nm = seg_plain(s1_nm, pack, meta)

    def body(_, M_flat):
        Y = _tc_matmul(M_flat.reshape(n_pad, m), Wt, blk=1024)
        return seg_bias_relu(Y, pack, meta, b_nm)

    M0 = jnp.zeros((n_pad * m,), jnp.float32)
    M_fin = lax.fori_loop(0, fw_mitr, body, M0)
    return M_fin.reshape(n_pad, m)[:n].T


# speedup vs baseline: 2.2327x; 2.2327x over previous
import jax
import jax.numpy as jnp
from jax import lax
from jax.experimental import pallas as pl
from jax.experimental.pallas import tpu as pltpu
from jax.experimental.pallas import tpu_sc as plsc

NC, NS, LANES = 2, 16, 16
NW = NC * NS
CHUNK = 64
NGP = CHUNK // LANES
PACKW = 3 * CHUNK + LANES


def _project_linf(W, v):
    m = W.shape[1]
    absW = jnp.abs(W)
    s = jnp.sum(absW, axis=1, keepdims=True)
    u = jnp.sort(absW, axis=1)[:, ::-1]
    css = jnp.cumsum(u, axis=1)
    idx = jnp.arange(1, m + 1)
    cond = u - (css - v) / idx.astype(W.dtype) > 0
    rho = jnp.max(jnp.where(cond, idx, 0), axis=1, keepdims=True)
    theta = (jnp.take_along_axis(css, rho - 1, axis=1) - v) / rho.astype(W.dtype)
    proj = jnp.sign(W) * jnp.maximum(absW - theta, 0.0)
    return jnp.where(s > v, proj, W)


def _mm_body(m_ref, w_ref, o_ref):
    o_ref[...] = jnp.dot(m_ref[...], w_ref[...], preferred_element_type=jnp.float32)


def _tc_matmul(M, Wt, blk):
    n_pad, m = M.shape
    return pl.pallas_call(
        _mm_body,
        grid=(n_pad // blk,),
        in_specs=[
            pl.BlockSpec((blk, m), lambda i: (i, 0)),
            pl.BlockSpec((m, m), lambda i: (0, 0)),
        ],
        out_specs=pl.BlockSpec((blk, m), lambda i: (i, 0)),
        out_shape=jax.ShapeDtypeStruct((n_pad, m), jnp.float32),
    )(M, Wt)


def _sload(ref, i):
    return ref[pl.ds(i, LANES)][0]


def _make_seg_kernel(n_pad, m, rows, with_bias):
    mesh = plsc.VectorSubcoreMesh(
        core_axis_name="c", subcore_axis_name="s", num_cores=NC, num_subcores=NS
    )
    acc_words = rows * m

    def body(y_hbm, pack_hbm, meta_hbm, *rest):
        if with_bias:
            (bias_hbm, out_hbm, acc, gb0, gb1, ib0, ib1, metav,
             sg0, sg1, si0, si1) = rest
        else:
            (out_hbm, acc, gb0, gb1, ib0, ib1, metav,
             sg0, sg1, si0, si1) = rest
        c_id = lax.axis_index("c")
        s_id = lax.axis_index("s")
        wid = c_id * NS + s_id
        base_row = wid * rows
        iota = jnp.arange(LANES, dtype=jnp.int32)

        pltpu.sync_copy(meta_hbm, metav)
        start = _sload(metav, wid)
        end = _sload(metav, NW + wid)
        start_row = start // CHUNK
        nch = (end - start_row * CHUNK + (CHUNK - 1)) // CHUNK
        nblk2 = jnp.maximum((nch + 1) // 2, 1)

        def fire_idx(row, ib, sem):
            pltpu.async_copy(pack_hbm.at[pl.ds(row, 1)], ib, sem)

        def wait_idx(ib, sem):
            pltpu.make_async_copy(pack_hbm.at[pl.ds(0, 1)], ib, sem).wait()

        def fire_g(ib, gb, sem):
            pltpu.async_copy(y_hbm.at[ib.at[0, pl.ds(0, CHUNK)]], gb, sem)

        def wait_g(ib, gb, sem):
            pltpu.make_async_copy(y_hbm.at[ib.at[0, pl.ds(0, CHUNK)]], gb, sem).wait()

        if with_bias:
            pltpu.sync_copy(bias_hbm.at[pl.ds(base_row * m, acc_words)], acc)
        else:
            zeros16 = jnp.zeros((LANES,), jnp.float32)

            def zero_body(i, _):
                acc[pl.ds(i * LANES, LANES)] = zeros16
                return 0

            lax.fori_loop(0, acc_words // LANES, zero_body, 0)

        def process(off, ib, gb):
            def edge_body(e):
                eid = off + e
                valid = (eid >= start) & (eid < end)
                dv = ib[0, pl.ds(CHUNK + e, LANES)][0]
                wv0 = plsc.bitcast(ib[0, pl.ds(2 * CHUNK + e, LANES)], jnp.float32)[0]
                d = jnp.where(valid, dv - base_row, 0)
                w = jnp.where(valid, wv0, 0.0)
                wvec = jnp.broadcast_to(w, (LANES,))
                ab = d * m
                for j in range(m // LANES):
                    x = gb[e, pl.ds(j * LANES, LANES)]
                    plsc.addupdate(acc.at[pl.ds(ab + j * LANES, LANES)], x * wvec)

            plsc.parallel_loop(0, CHUNK, 1, unroll=2)(edge_body)

        fire_idx(start_row, ib0, si0)
        wait_idx(ib0, si0)
        fire_g(ib0, gb0, sg0)
        fire_idx(start_row + 1, ib1, si1)

        def blk_body(bi, _):
            for b, (ib_c, gb_c, si_c, sg_c, ib_n, gb_n, si_n, sg_n) in enumerate(
                ((ib0, gb0, si0, sg0, ib1, gb1, si1, sg1),
                 (ib1, gb1, si1, sg1, ib0, gb0, si0, sg0))
            ):
                cc = bi * 2 + b
                wait_g(ib_c, gb_c, sg_c)
                wait_idx(ib_n, si_n)
                fire_g(ib_n, gb_n, sg_n)
                process((start_row + cc) * CHUNK, ib_c, gb_c)
                fire_idx(start_row + cc + 2, ib_c, si_c)
            return 0

        lax.fori_loop(0, nblk2, blk_body, 0)

        wait_g(ib0, gb0, sg0)
        wait_idx(ib1, si1)

        if with_bias:
            def relu_body(i, _):
                v = acc[pl.ds(i * LANES, LANES)]
                acc[pl.ds(i * LANES, LANES)] = jnp.maximum(v, 0.0)
                return 0

            lax.fori_loop(0, acc_words // LANES, relu_body, 0)

        pltpu.sync_copy(acc, out_hbm.at[pl.ds(base_row * m, acc_words)])

    scratch = [
        pltpu.VMEM((acc_words,), jnp.float32),
        pltpu.VMEM((CHUNK, m), jnp.float32),
        pltpu.VMEM((CHUNK, m), jnp.float32),
        pltpu.VMEM((1, PACKW), jnp.int32),
        pltpu.VMEM((1, PACKW), jnp.int32),
        pltpu.VMEM((NW * 2 + LANES,), jnp.int32),
        pltpu.SemaphoreType.DMA,
        pltpu.SemaphoreType.DMA,
        pltpu.SemaphoreType.DMA,
        pltpu.SemaphoreType.DMA,
    ]
    return pl.kernel(
        body,
        out_type=jax.ShapeDtypeStruct((n_pad * m,), jnp.float32),
        mesh=mesh,
        scratch_types=scratch,
        compiler_params=pltpu.CompilerParams(
            use_tc_tiling_on_sc=False, needs_layout_passes=False
        ),
    )


def kernel(X_0, edge_index, edge_weight, U, W, Omega_1, fw_mitr):
    m, n = X_0.shape
    p = U.shape[0]
    E = edge_index.shape[1]
    kappa, A_rho = 0.99, 1.0

    rows = ((n + NW - 1) // NW + 7) // 8 * 8
    n_pad = NW * rows
    e_pad = ((E + CHUNK - 1) // CHUNK) * CHUNK + 8 * CHUNK

    W_p = _project_linf(W, kappa / A_rho)
    Wt = W_p.T

    src = edge_index[0].astype(jnp.int32)
    dst = edge_index[1].astype(jnp.int32)
    order = jnp.argsort(dst)
    src_s = jnp.concatenate([src[order], jnp.zeros((e_pad - E,), jnp.int32)])
    dst_s = jnp.concatenate([dst[order], jnp.zeros((e_pad - E,), jnp.int32)])
    w_s = jnp.concatenate(
        [edge_weight[order].astype(jnp.float32), jnp.zeros((e_pad - E,), jnp.float32)]
    )
    nrow = e_pad // CHUNK
    pack = jnp.concatenate(
        [
            src_s.reshape(nrow, CHUNK),
            dst_s.reshape(nrow, CHUNK),
            lax.bitcast_convert_type(w_s, jnp.int32).reshape(nrow, CHUNK),
            jnp.zeros((nrow, LANES), jnp.int32),
        ],
        axis=1,
    )
    bounds = jnp.searchsorted(
        dst_s[:E], jnp.arange(NW + 1, dtype=jnp.int32) * rows
    ).astype(jnp.int32)
    meta = jnp.concatenate(
        [bounds[:NW], bounds[1 : NW + 1], jnp.zeros((LANES,), jnp.int32)]
    )

    seg_plain = _make_seg_kernel(n_pad, m, rows, with_bias=False)
    seg_bias_relu = _make_seg_kernel(n_pad, m, rows, with_bias=True)

    ut_pad = jnp.zeros((n_pad, p), jnp.float32).at[:n].set(U.T)
    s1_nm = _tc_matmul(ut_pad, Omega_1.T, blk=1024)
    b_nm = seg_plain(s1_nm, pack, meta)

    def body(_, M_flat):
        Y = _tc_matmul(M_flat.reshape(n_pad, m), Wt, blk=1024)
        return seg_bias_relu(Y, pack, meta, b_nm)

    M0 = jnp.zeros((n_pad * m,), jnp.float32)
    M_fin = lax.fori_loop(0, fw_mitr, body, M0)
    return M_fin.reshape(n_pad, m)[:n].T


# speedup vs baseline: 2.2666x; 1.0152x over previous
import jax
import jax.numpy as jnp
from jax import lax
from jax.experimental import pallas as pl
from jax.experimental.pallas import tpu as pltpu
from jax.experimental.pallas import tpu_sc as plsc

NC, NS, LANES = 2, 16, 16
NW = NC * NS
CHUNK = 64
NGP = CHUNK // LANES
PACKW = 3 * CHUNK + LANES


def _project_linf(W, v):
    m = W.shape[1]
    absW = jnp.abs(W)
    s = jnp.sum(absW, axis=1, keepdims=True)
    u = jnp.sort(absW, axis=1)[:, ::-1]
    css = jnp.cumsum(u, axis=1)
    idx = jnp.arange(1, m + 1)
    cond = u - (css - v) / idx.astype(W.dtype) > 0
    rho = jnp.max(jnp.where(cond, idx, 0), axis=1, keepdims=True)
    theta = (jnp.take_along_axis(css, rho - 1, axis=1) - v) / rho.astype(W.dtype)
    proj = jnp.sign(W) * jnp.maximum(absW - theta, 0.0)
    return jnp.where(s > v, proj, W)


def _mm_body(m_ref, w_ref, o_ref):
    o_ref[...] = jnp.dot(m_ref[...], w_ref[...], preferred_element_type=jnp.float32)


def _tc_matmul(M, Wt, blk):
    n_pad, m = M.shape
    return pl.pallas_call(
        _mm_body,
        grid=(n_pad // blk,),
        in_specs=[
            pl.BlockSpec((blk, m), lambda i: (i, 0)),
            pl.BlockSpec((m, m), lambda i: (0, 0)),
        ],
        out_specs=pl.BlockSpec((blk, m), lambda i: (i, 0)),
        out_shape=jax.ShapeDtypeStruct((n_pad, m), jnp.float32),
    )(M, Wt)


def _sload(ref, i):
    return ref[pl.ds(i, LANES)][0]


def _make_seg_kernel(n_pad, m, rows, with_bias):
    mesh = plsc.VectorSubcoreMesh(
        core_axis_name="c", subcore_axis_name="s", num_cores=NC, num_subcores=NS
    )
    acc_words = rows * m

    def body(y_hbm, pack_hbm, meta_hbm, *rest):
        if with_bias:
            (bias_hbm, out_hbm, acc, gb0, gb1, ib0, ib1, metav,
             sg0, sg1, si0, si1) = rest
        else:
            (out_hbm, acc, gb0, gb1, ib0, ib1, metav,
             sg0, sg1, si0, si1) = rest
        c_id = lax.axis_index("c")
        s_id = lax.axis_index("s")
        wid = c_id * NS + s_id
        base_row = wid * rows
        iota = jnp.arange(LANES, dtype=jnp.int32)

        pltpu.sync_copy(meta_hbm, metav)
        start = _sload(metav, wid)
        end = _sload(metav, NW + wid)
        start_row = start // CHUNK
        nch = (end - start_row * CHUNK + (CHUNK - 1)) // CHUNK
        nblk2 = jnp.maximum((nch + 1) // 2, 1)

        def fire_idx(row, ib, sem):
            pltpu.async_copy(pack_hbm.at[pl.ds(row, 1)], ib, sem)

        def wait_idx(ib, sem):
            pltpu.make_async_copy(pack_hbm.at[pl.ds(0, 1)], ib, sem).wait()

        def fire_g(ib, gb, sem):
            pltpu.async_copy(y_hbm.at[ib.at[0, pl.ds(0, CHUNK)]], gb, sem)

        def wait_g(ib, gb, sem):
            pltpu.make_async_copy(y_hbm.at[ib.at[0, pl.ds(0, CHUNK)]], gb, sem).wait()

        if with_bias:
            pltpu.sync_copy(bias_hbm.at[pl.ds(base_row * m, acc_words)], acc)
        else:
            zeros16 = jnp.zeros((LANES,), jnp.float32)

            def zero_body(i, _):
                acc[pl.ds(i * LANES, LANES)] = zeros16
                return 0

            lax.fori_loop(0, acc_words // LANES, zero_body, 0)

        def process(off, ib, gb):
            def edge_body(e):
                eid = off + e
                valid = (eid >= start) & (eid < end)
                dv = ib[0, pl.ds(CHUNK + e, LANES)][0]
                wv0 = plsc.bitcast(ib[0, pl.ds(2 * CHUNK + e, LANES)], jnp.float32)[0]
                d = jnp.where(valid, dv - base_row, 0)
                w = jnp.where(valid, wv0, 0.0)
                wvec = jnp.broadcast_to(w, (LANES,))
                ab = d * m
                for j in range(m // LANES):
                    x = gb[e, pl.ds(j * LANES, LANES)]
                    plsc.addupdate(acc.at[pl.ds(ab + j * LANES, LANES)], x * wvec)

            plsc.parallel_loop(0, CHUNK, 1, unroll=4)(edge_body)

        fire_idx(start_row, ib0, si0)
        wait_idx(ib0, si0)
        fire_g(ib0, gb0, sg0)
        fire_idx(start_row + 1, ib1, si1)

        def blk_body(bi, _):
            for b, (ib_c, gb_c, si_c, sg_c, ib_n, gb_n, si_n, sg_n) in enumerate(
                ((ib0, gb0, si0, sg0, ib1, gb1, si1, sg1),
                 (ib1, gb1, si1, sg1, ib0, gb0, si0, sg0))
            ):
                cc = bi * 2 + b
                wait_g(ib_c, gb_c, sg_c)
                wait_idx(ib_n, si_n)
                fire_g(ib_n, gb_n, sg_n)
                process((start_row + cc) * CHUNK, ib_c, gb_c)
                fire_idx(start_row + cc + 2, ib_c, si_c)
            return 0

        lax.fori_loop(0, nblk2, blk_body, 0)

        wait_g(ib0, gb0, sg0)
        wait_idx(ib1, si1)

        if with_bias:
            def relu_body(i, _):
                v = acc[pl.ds(i * LANES, LANES)]
                acc[pl.ds(i * LANES, LANES)] = jnp.maximum(v, 0.0)
                return 0

            lax.fori_loop(0, acc_words // LANES, relu_body, 0)

        pltpu.sync_copy(acc, out_hbm.at[pl.ds(base_row * m, acc_words)])

    scratch = [
        pltpu.VMEM((acc_words,), jnp.float32),
        pltpu.VMEM((CHUNK, m), jnp.float32),
        pltpu.VMEM((CHUNK, m), jnp.float32),
        pltpu.VMEM((1, PACKW), jnp.int32),
        pltpu.VMEM((1, PACKW), jnp.int32),
        pltpu.VMEM((NW * 2 + LANES,), jnp.int32),
        pltpu.SemaphoreType.DMA,
        pltpu.SemaphoreType.DMA,
        pltpu.SemaphoreType.DMA,
        pltpu.SemaphoreType.DMA,
    ]
    return pl.kernel(
        body,
        out_type=jax.ShapeDtypeStruct((n_pad * m,), jnp.float32),
        mesh=mesh,
        scratch_types=scratch,
        compiler_params=pltpu.CompilerParams(
            use_tc_tiling_on_sc=False, needs_layout_passes=False
        ),
    )


def kernel(X_0, edge_index, edge_weight, U, W, Omega_1, fw_mitr):
    m, n = X_0.shape
    p = U.shape[0]
    E = edge_index.shape[1]
    kappa, A_rho = 0.99, 1.0

    rows = ((n + NW - 1) // NW + 7) // 8 * 8
    n_pad = NW * rows
    e_pad = ((E + CHUNK - 1) // CHUNK) * CHUNK + 8 * CHUNK

    W_p = _project_linf(W, kappa / A_rho)
    Wt = W_p.T

    src = edge_index[0].astype(jnp.int32)
    dst = edge_index[1].astype(jnp.int32)
    order = jnp.argsort(dst)
    src_s = jnp.concatenate([src[order], jnp.zeros((e_pad - E,), jnp.int32)])
    dst_s = jnp.concatenate([dst[order], jnp.zeros((e_pad - E,), jnp.int32)])
    w_s = jnp.concatenate(
        [edge_weight[order].astype(jnp.float32), jnp.zeros((e_pad - E,), jnp.float32)]
    )
    nrow = e_pad // CHUNK
    pack = jnp.concatenate(
        [
            src_s.reshape(nrow, CHUNK),
            dst_s.reshape(nrow, CHUNK),
            lax.bitcast_convert_type(w_s, jnp.int32).reshape(nrow, CHUNK),
            jnp.zeros((nrow, LANES), jnp.int32),
        ],
        axis=1,
    )
    bounds = jnp.searchsorted(
        dst_s[:E], jnp.arange(NW + 1, dtype=jnp.int32) * rows
    ).astype(jnp.int32)
    meta = jnp.concatenate(
        [bounds[:NW], bounds[1 : NW + 1], jnp.zeros((LANES,), jnp.int32)]
    )

    seg_plain = _make_seg_kernel(n_pad, m, rows, with_bias=False)
    seg_bias_relu = _make_seg_kernel(n_pad, m, rows, with_bias=True)

    ut_pad = jnp.zeros((n_pad, p), jnp.float32).at[:n].set(U.T)
    s1_nm = _tc_matmul(ut_pad, Omega_1.T, blk=1024)
    b_nm = seg_plain(s1_nm, pack, meta)

    def body(_, M_flat):
        Y = _tc_matmul(M_flat.reshape(n_pad, m), Wt, blk=1024)
        return seg_bias_relu(Y, pack, meta, b_nm)

    M0 = jnp.zeros((n_pad * m,), jnp.float32)
    M_fin = lax.fori_loop(0, fw_mitr, body, M0)
    return M_fin.reshape(n_pad, m)[:n].T
